# baseline (speedup 1.0000x reference)
import functools

import jax
import jax.numpy as jnp
from jax import lax
from jax.experimental import pallas as pl
from jax.experimental.pallas import tpu as pltpu
from jax.experimental.pallas import tpu_sc as plsc

DIM = 32
IT = 0.01
SC_OFF = 2 * IT * 0.5772156649015329
LN2 = 0.6931471805599453
SQRT2 = 1.4142135623730951


_RW = 131072
_ALIGN = 128


def _relayout_body(band, a_ref, b_ref, c_ref, ta_ref, tb_ref, tc_ref,
                   oa_ref, ob_ref, oc_ref,
                   rb0, rb1, rb2, rb3, rsem, wsem, tsem):
    del tsem
    N = a_ref.shape[2]
    NA = (N // _ALIGN) * _ALIGN
    NP = (N + _ALIGN - 1) // _ALIGN * _ALIGN
    TS = ta_ref.shape[0]
    rbufs = (rb0, rb1, rb2, rb3)
    NB = len(rbufs)

    n_full = NA // _RW
    chunks = [(k * _RW, _RW) for k in range(n_full)]
    if NA > n_full * _RW:
        chunks.append((n_full * _RW, NA - n_full * _RW))

    plans = [(a_ref, ta_ref, oa_ref), (b_ref, tb_ref, ob_ref),
             (c_ref, tc_ref, oc_ref)]
    work = []
    for t_ref, _, o_ref in plans:
        for c0, ln in chunks:
            work.append((t_ref, o_ref, band, c0, ln))

    def read_desc(i):
        t_ref, o_ref, band, c0, ln = work[i]
        buf = rbufs[i % NB]
        return pltpu.make_async_copy(
            t_ref.at[0, pl.ds(band * 8, 8), pl.ds(c0, ln)],
            buf.at[:, pl.ds(0, ln)], rsem[i % NB])

    def write_descs(i):
        t_ref, o_ref, band, c0, ln = work[i]
        buf = rbufs[i % NB]
        return [pltpu.make_async_copy(
                    buf.at[r, pl.ds(0, ln)],
                    o_ref.at[pl.ds(r * NP + c0, ln)],
                    wsem[i % NB])
                for r in range(8)]

    n = len(work)
    for i in range(n):
        if i >= NB:
            for d in write_descs(i - NB):
                d.wait()
        read_desc(i).start()
        if i >= 2:
            read_desc(i - 2).wait()
            for d in write_descs(i - 2):
                d.start()
    for j in (n - 2, n - 1):
        if j >= 0:
            read_desc(j).wait()
            for d in write_descs(j):
                d.start()
    lo = max(0, n - NB)
    for j in range(lo, n):
        for d in write_descs(j):
            d.wait()

    for _, tl_ref, o_ref in plans:
        pltpu.make_async_copy(tl_ref, rb0.at[0, pl.ds(0, TS)], rsem[0]).start()
        pltpu.make_async_copy(tl_ref, rb0.at[0, pl.ds(0, TS)], rsem[0]).wait()
        pltpu.async_copy(rb0.at[0, pl.ds(0, TS)],
                         o_ref.at[pl.ds(8 * NP, TS)], rsem[0]).wait()


def _relayout(band, t1, t2, t3, tl1, tl2, tl3):
    N = t1.shape[2]
    NP = (N + _ALIGN - 1) // _ALIGN * _ALIGN
    TS = tl1.shape[0]
    any_spec = pl.BlockSpec(memory_space=pltpu.MemorySpace.HBM)
    return pl.pallas_call(
        functools.partial(_relayout_body, band),
        in_specs=[any_spec] * 6,
        out_specs=[any_spec, any_spec, any_spec],
        out_shape=[jax.ShapeDtypeStruct((8 * NP + TS,), jnp.float32)] * 3,
        scratch_shapes=[
            pltpu.VMEM((8, _RW), jnp.float32),
            pltpu.VMEM((8, _RW), jnp.float32),
            pltpu.VMEM((8, _RW), jnp.float32),
            pltpu.VMEM((8, _RW), jnp.float32),
            [pltpu.SemaphoreType.DMA] * 4,
            [pltpu.SemaphoreType.DMA] * 4,
            pltpu.SemaphoreType.DMA,
        ],
    )(t1, t2, t3, tl1, tl2, tl3)


def _vlog(x):
    xi = lax.bitcast_convert_type(x, jnp.int32)
    e = lax.shift_right_arithmetic(xi, 23) - 127
    m = lax.bitcast_convert_type(
        jnp.bitwise_or(jnp.bitwise_and(xi, 0x7FFFFF), 0x3F800000), jnp.float32)
    big = m > SQRT2
    m = jnp.where(big, m * 0.5, m)
    e = jnp.where(big, e + 1, e)
    s = (m - 1.0) / (m + 1.0)
    z = s * s
    p = 2.0 * s * (1.0 + z * (1.0 / 3.0 + z * (0.2 + z * (1.0 / 7.0 + z / 9.0))))
    return p + e.astype(jnp.float32) * LN2


def _log1p(t):
    return _vlog(1.0 + t)


def _softplus(x):
    return jnp.maximum(x, 0.0) + _log1p(jnp.exp(-jnp.abs(x)))


def _logaddexp(a, b):
    return jnp.maximum(a, b) + _log1p(jnp.exp(-jnp.abs(a - b)))


def _term(c1, e1b, b1, c2, e2b, b2):
    w1 = _softplus(e1b) * 0.5
    w2 = _softplus(e2b) * 0.5
    min1 = c1 - w1
    max1 = c1 + w1
    min2 = c2 - w2
    max2 = c2 + w2
    bin_vec = 1.0 / ((1.0 + jnp.exp(-b1)) * (1.0 + jnp.exp(-b2)))
    meet_min = IT * _logaddexp(min1 / IT, min2 / IT)
    meet_max = -IT * _logaddexp(-max1 / IT, -max2 / IT)
    meet_min = jnp.maximum(meet_min, jnp.maximum(min1, min2))
    meet_max = jnp.minimum(meet_max, jnp.minimum(max1, max2))
    lv_meet = _vlog(_softplus(meet_max - meet_min - SC_OFF) + 1e-20)
    lv_rhs = _vlog(_softplus(max2 - min2 - SC_OFF) + 1e-20)
    return (lv_meet - lv_rhs) * bin_vec


def _make_sc_kernel(B, N, n_prev):
    DIMB = 8
    NP = (N + _ALIGN - 1) // _ALIGN * _ALIGN
    NA = (N // _ALIGN) * _ALIGN
    TR = N - NA
    stash_base = DIMB * NP
    info = plsc.get_sparse_core_info()
    NC, NS = info.num_cores, info.num_subcores
    NW = NC * NS
    b_per_w = B // NW
    CH = 256
    n_ch = b_per_w // CH
    NG = CH // 16

    buf_set = [
        pltpu.VMEM((CH,), jnp.int32),
        pltpu.VMEM((CH,), jnp.int32),
        pltpu.VMEM((DIMB * CH,), jnp.int32),
        pltpu.VMEM((DIMB * CH,), jnp.int32),
        [pltpu.VMEM((DIMB * CH,), jnp.float32) for _ in range(6)],
        pltpu.SemaphoreType.DMA,
    ]

    @functools.partial(
        pl.kernel,
        mesh=plsc.VectorSubcoreMesh(core_axis_name="c", subcore_axis_name="s"),
        compiler_params=pltpu.CompilerParams(use_tc_tiling_on_sc=False),
        out_type=jax.ShapeDtypeStruct((B,), jnp.float32),
        scratch_types=[buf_set, buf_set, pltpu.VMEM((CH,), jnp.float32),
                       [pltpu.VMEM((CH,), jnp.float32)] * n_prev],
    )
    def sc_k(idx1_hbm, idx2_hbm, t1, t2, t3, *rest):
        prevs = rest[:n_prev]
        out_hbm = rest[n_prev]
        set0, set1, ob, pbufs = rest[n_prev + 1:]
        wid = lax.axis_index("s") * NC + lax.axis_index("c")
        base = wid * b_per_w
        sets = (set0, set1)

        def load_and_fire(ch, st):
            i1_v, i2_v, ib1, ib2, dsts, sem = st
            off = base + ch * CH
            pltpu.sync_copy(idx1_hbm.at[pl.ds(off, CH)], i1_v)
            pltpu.sync_copy(idx2_hbm.at[pl.ds(off, CH)], i2_v)

            def build(j, _):
                v1 = i1_v[pl.ds(j * 16, 16)]
                v2 = i2_v[pl.ds(j * 16, 16)]

                def per_d(d, _):
                    dn = d * NP
                    a1 = jnp.where(v1 < NA, v1 + dn,
                                   v1 + (stash_base + d * TR - NA))
                    a2 = jnp.where(v2 < NA, v2 + dn,
                                   v2 + (stash_base + d * TR - NA))
                    ib1[pl.ds(d * CH + j * 16, 16)] = a1
                    ib2[pl.ds(d * CH + j * 16, 16)] = a2
                    return 0

                return lax.fori_loop(0, DIMB, per_d, 0)

            lax.fori_loop(0, NG, build, 0)
            plan = ((t1, ib1), (t2, ib1), (t3, ib1),
                    (t1, ib2), (t2, ib2), (t3, ib2))
            return [pltpu.async_copy(tab.at[ib], dsts[k], sem)
                    for k, (tab, ib) in enumerate(plan)]

        def compute_and_store(ch, st, cps):
            _, _, _, _, dsts, sem = st
            d0, d1, d2, d3, d4, d5 = dsts
            off = base + ch * CH
            for k in range(n_prev):
                pltpu.sync_copy(prevs[k].at[pl.ds(off, CH)], pbufs[k])
            for cp in cps:
                cp.wait()

            def group(pg, _):
                def per_d(d, acc):
                    q = d * CH + pg * 16
                    return acc + _term(d0[pl.ds(q, 16)], d1[pl.ds(q, 16)],
                                       d2[pl.ds(q, 16)], d3[pl.ds(q, 16)],
                                       d4[pl.ds(q, 16)], d5[pl.ds(q, 16)])

                acc = lax.fori_loop(0, DIMB, per_d, jnp.zeros(16, jnp.float32))
                for k in range(n_prev):
                    acc = acc + pbufs[k][pl.ds(pg * 16, 16)]
                ob[pl.ds(pg * 16, 16)] = acc
                return 0

            lax.fori_loop(0, NG, group, 0)
            pltpu.sync_copy(ob, out_hbm.at[pl.ds(off, CH)])

        pend = load_and_fire(0, sets[0])
        for ch in range(n_ch):
            nxt = None
            if ch + 1 < n_ch:
                nxt = load_and_fire(ch + 1, sets[(ch + 1) % 2])
            compute_and_store(ch, sets[ch % 2], pend)
            pend = nxt

    return sc_k


def kernel(idx1, idx2, emb1, emb2, embs1_w, embs2_w, bins_w):
    del emb1, emb2
    B = idx1.shape[0]
    N = embs1_w.shape[0]
    NA = (N // _ALIGN) * _ALIGN
    views = [jnp.transpose(t)[None] for t in (embs1_w, embs2_w, bins_w)]
    tails = [jnp.transpose(t[NA:]).reshape(-1)
             for t in (embs1_w, embs2_w, bins_w)]
    TR = N - NA
    n_bands = DIM // 8
    partials = []
    for band in range(n_bands):
        band_tails = [tl[pl.ds(band * 8 * TR, 8 * TR)]
                      if False else tl[band * 8 * TR:(band + 1) * 8 * TR]
                      for tl in tails]
        f1, f2, f3 = _relayout(band, *views, *band_tails)
        sck = _make_sc_kernel(B, N, len(partials) if band == n_bands - 1 else 0)
        if band == n_bands - 1:
            out = sck(idx1, idx2, f1, f2, f3, *partials)
        else:
            partials.append(sck(idx1, idx2, f1, f2, f3))
    return out


# speedup vs baseline: 1.0314x; 1.0314x over previous
import functools

import jax
import jax.numpy as jnp
from jax import lax
from jax.experimental import pallas as pl
from jax.experimental.pallas import tpu as pltpu
from jax.experimental.pallas import tpu_sc as plsc

DIM = 32
IT = 0.01
SC_OFF = 2 * IT * 0.5772156649015329
LN2 = 0.6931471805599453
SQRT2 = 1.4142135623730951


_RW = 131072
_ALIGN = 128


def _relayout_body(band, a_ref, b_ref, c_ref, ta_ref, tb_ref, tc_ref,
                   oa_ref, ob_ref, oc_ref,
                   rb0, rb1, rb2, rb3, rsem, wsem, tsem):
    del tsem
    N = a_ref.shape[2]
    NA = (N // _ALIGN) * _ALIGN
    NP = (N + _ALIGN - 1) // _ALIGN * _ALIGN
    TS = ta_ref.shape[0]
    rbufs = (rb0, rb1, rb2, rb3)
    NB = len(rbufs)

    n_full = NA // _RW
    chunks = [(k * _RW, _RW) for k in range(n_full)]
    if NA > n_full * _RW:
        chunks.append((n_full * _RW, NA - n_full * _RW))

    plans = [(a_ref, ta_ref, oa_ref), (b_ref, tb_ref, ob_ref),
             (c_ref, tc_ref, oc_ref)]
    work = []
    for t_ref, _, o_ref in plans:
        for c0, ln in chunks:
            work.append((t_ref, o_ref, band, c0, ln))

    def read_desc(i):
        t_ref, o_ref, band, c0, ln = work[i]
        buf = rbufs[i % NB]
        return pltpu.make_async_copy(
            t_ref.at[0, pl.ds(band * 8, 8), pl.ds(c0, ln)],
            buf.at[:, pl.ds(0, ln)], rsem[i % NB])

    def write_descs(i):
        t_ref, o_ref, band, c0, ln = work[i]
        buf = rbufs[i % NB]
        return [pltpu.make_async_copy(
                    buf.at[r, pl.ds(0, ln)],
                    o_ref.at[pl.ds(r * NP + c0, ln)],
                    wsem[i % NB])
                for r in range(8)]

    n = len(work)
    for i in range(n):
        if i >= NB:
            for d in write_descs(i - NB):
                d.wait()
        read_desc(i).start()
        if i >= 2:
            read_desc(i - 2).wait()
            for d in write_descs(i - 2):
                d.start()
    for j in (n - 2, n - 1):
        if j >= 0:
            read_desc(j).wait()
            for d in write_descs(j):
                d.start()
    lo = max(0, n - NB)
    for j in range(lo, n):
        for d in write_descs(j):
            d.wait()

    for _, tl_ref, o_ref in plans:
        pltpu.make_async_copy(tl_ref, rb0.at[0, pl.ds(0, TS)], rsem[0]).start()
        pltpu.make_async_copy(tl_ref, rb0.at[0, pl.ds(0, TS)], rsem[0]).wait()
        pltpu.async_copy(rb0.at[0, pl.ds(0, TS)],
                         o_ref.at[pl.ds(8 * NP, TS)], rsem[0]).wait()


def _relayout(band, t1, t2, t3, tl1, tl2, tl3):
    N = t1.shape[2]
    NP = (N + _ALIGN - 1) // _ALIGN * _ALIGN
    TS = tl1.shape[0]
    any_spec = pl.BlockSpec(memory_space=pltpu.MemorySpace.HBM)
    return pl.pallas_call(
        functools.partial(_relayout_body, band),
        in_specs=[any_spec] * 6,
        out_specs=[any_spec, any_spec, any_spec],
        out_shape=[jax.ShapeDtypeStruct((8 * NP + TS,), jnp.float32)] * 3,
        scratch_shapes=[
            pltpu.VMEM((8, _RW), jnp.float32),
            pltpu.VMEM((8, _RW), jnp.float32),
            pltpu.VMEM((8, _RW), jnp.float32),
            pltpu.VMEM((8, _RW), jnp.float32),
            [pltpu.SemaphoreType.DMA] * 4,
            [pltpu.SemaphoreType.DMA] * 4,
            pltpu.SemaphoreType.DMA,
        ],
    )(t1, t2, t3, tl1, tl2, tl3)


def _vlog(x):
    xi = lax.bitcast_convert_type(x, jnp.int32)
    e = lax.shift_right_arithmetic(xi, 23) - 127
    m = lax.bitcast_convert_type(
        jnp.bitwise_or(jnp.bitwise_and(xi, 0x7FFFFF), 0x3F800000), jnp.float32)
    big = m > SQRT2
    m = jnp.where(big, m * 0.5, m)
    e = jnp.where(big, e + 1, e)
    s = (m - 1.0) / (m + 1.0)
    z = s * s
    p = 2.0 * s * (1.0 + z * (1.0 / 3.0 + z * (0.2 + z * (1.0 / 7.0 + z / 9.0))))
    return p + e.astype(jnp.float32) * LN2


def _log1p(t):
    return _vlog(1.0 + t)


def _softplus(x):
    return jnp.maximum(x, 0.0) + _log1p(jnp.exp(-jnp.abs(x)))


def _logaddexp(a, b):
    return jnp.maximum(a, b) + _log1p(jnp.exp(-jnp.abs(a - b)))


def _term(c1, e1b, b1, c2, e2b, b2):
    w1 = _softplus(e1b) * 0.5
    w2 = _softplus(e2b) * 0.5
    min1 = c1 - w1
    max1 = c1 + w1
    min2 = c2 - w2
    max2 = c2 + w2
    bin_vec = 1.0 / ((1.0 + jnp.exp(-b1)) * (1.0 + jnp.exp(-b2)))
    meet_min = IT * _logaddexp(min1 / IT, min2 / IT)
    meet_max = -IT * _logaddexp(-max1 / IT, -max2 / IT)
    meet_min = jnp.maximum(meet_min, jnp.maximum(min1, min2))
    meet_max = jnp.minimum(meet_max, jnp.minimum(max1, max2))
    lv_meet = _vlog(_softplus(meet_max - meet_min - SC_OFF) + 1e-20)
    lv_rhs = _vlog(_softplus(max2 - min2 - SC_OFF) + 1e-20)
    return (lv_meet - lv_rhs) * bin_vec


def _make_sc_kernel(B, N, n_prev):
    DIMB = 8
    NP = (N + _ALIGN - 1) // _ALIGN * _ALIGN
    NA = (N // _ALIGN) * _ALIGN
    TR = N - NA
    stash_base = DIMB * NP
    info = plsc.get_sparse_core_info()
    NC, NS = info.num_cores, info.num_subcores
    NW = NC * NS
    b_per_w = B // NW
    CH = 128
    n_ch = b_per_w // CH
    NG = CH // 16

    buf_set = [
        pltpu.VMEM((CH,), jnp.int32),
        pltpu.VMEM((CH,), jnp.int32),
        pltpu.VMEM((DIMB * CH,), jnp.int32),
        pltpu.VMEM((DIMB * CH,), jnp.int32),
        [pltpu.VMEM((DIMB * CH,), jnp.float32) for _ in range(6)],
        pltpu.SemaphoreType.DMA,
    ]

    @functools.partial(
        pl.kernel,
        mesh=plsc.VectorSubcoreMesh(core_axis_name="c", subcore_axis_name="s"),
        compiler_params=pltpu.CompilerParams(use_tc_tiling_on_sc=False),
        out_type=jax.ShapeDtypeStruct((B,), jnp.float32),
        scratch_types=[buf_set, buf_set, pltpu.VMEM((CH,), jnp.float32),
                       [pltpu.VMEM((CH,), jnp.float32)] * n_prev],
    )
    def sc_k(idx1_hbm, idx2_hbm, t1, t2, t3, *rest):
        prevs = rest[:n_prev]
        out_hbm = rest[n_prev]
        set0, set1, ob, pbufs = rest[n_prev + 1:]
        wid = lax.axis_index("s") * NC + lax.axis_index("c")
        base = wid * b_per_w
        sets = (set0, set1)

        def load_and_fire(ch, st):
            i1_v, i2_v, ib1, ib2, dsts, sem = st
            off = base + ch * CH
            pltpu.sync_copy(idx1_hbm.at[pl.ds(off, CH)], i1_v)
            pltpu.sync_copy(idx2_hbm.at[pl.ds(off, CH)], i2_v)

            def build(j, _):
                v1 = i1_v[pl.ds(j * 16, 16)]
                v2 = i2_v[pl.ds(j * 16, 16)]

                def per_d(d, _):
                    dn = d * NP
                    a1 = jnp.where(v1 < NA, v1 + dn,
                                   v1 + (stash_base + d * TR - NA))
                    a2 = jnp.where(v2 < NA, v2 + dn,
                                   v2 + (stash_base + d * TR - NA))
                    ib1[pl.ds(d * CH + j * 16, 16)] = a1
                    ib2[pl.ds(d * CH + j * 16, 16)] = a2
                    return 0

                return lax.fori_loop(0, DIMB, per_d, 0)

            lax.fori_loop(0, NG, build, 0)
            plan = ((t1, ib1), (t2, ib1), (t3, ib1),
                    (t1, ib2), (t2, ib2), (t3, ib2))
            return [pltpu.async_copy(tab.at[ib], dsts[k], sem)
                    for k, (tab, ib) in enumerate(plan)]

        def compute_and_store(ch, st, cps):
            _, _, _, _, dsts, sem = st
            d0, d1, d2, d3, d4, d5 = dsts
            off = base + ch * CH
            for k in range(n_prev):
                pltpu.sync_copy(prevs[k].at[pl.ds(off, CH)], pbufs[k])
            for cp in cps:
                cp.wait()

            def group(pg, _):
                def per_d(d, acc):
                    q = d * CH + pg * 16
                    return acc + _term(d0[pl.ds(q, 16)], d1[pl.ds(q, 16)],
                                       d2[pl.ds(q, 16)], d3[pl.ds(q, 16)],
                                       d4[pl.ds(q, 16)], d5[pl.ds(q, 16)])

                acc = lax.fori_loop(0, DIMB, per_d, jnp.zeros(16, jnp.float32))
                for k in range(n_prev):
                    acc = acc + pbufs[k][pl.ds(pg * 16, 16)]
                ob[pl.ds(pg * 16, 16)] = acc
                return 0

            lax.fori_loop(0, NG, group, 0)
            pltpu.sync_copy(ob, out_hbm.at[pl.ds(off, CH)])

        pend = load_and_fire(0, sets[0])
        for ch in range(n_ch):
            nxt = None
            if ch + 1 < n_ch:
                nxt = load_and_fire(ch + 1, sets[(ch + 1) % 2])
            compute_and_store(ch, sets[ch % 2], pend)
            pend = nxt

    return sc_k


def kernel(idx1, idx2, emb1, emb2, embs1_w, embs2_w, bins_w):
    del emb1, emb2
    B = idx1.shape[0]
    N = embs1_w.shape[0]
    NA = (N // _ALIGN) * _ALIGN
    views = [jnp.transpose(t)[None] for t in (embs1_w, embs2_w, bins_w)]
    tails = [jnp.transpose(t[NA:]).reshape(-1)
             for t in (embs1_w, embs2_w, bins_w)]
    TR = N - NA
    n_bands = DIM // 8
    partials = []
    for band in range(n_bands):
        band_tails = [tl[pl.ds(band * 8 * TR, 8 * TR)]
                      if False else tl[band * 8 * TR:(band + 1) * 8 * TR]
                      for tl in tails]
        f1, f2, f3 = _relayout(band, *views, *band_tails)
        sck = _make_sc_kernel(B, N, len(partials) if band == n_bands - 1 else 0)
        if band == n_bands - 1:
            out = sck(idx1, idx2, f1, f2, f3, *partials)
        else:
            partials.append(sck(idx1, idx2, f1, f2, f3))
    return out
